# baseline (speedup 1.0000x reference)
import functools

import jax
import jax.numpy as jnp
from jax import lax
from jax.experimental import pallas as pl
from jax.experimental.pallas import tpu as pltpu
from jax.experimental.pallas import tpu_sc as plsc

_BETA = 0.25
_BT = 1024


def _tc_body(n_tok, total, z_ref, mask_ref, emb_ref, idx_ref, loss_ref,
             perp_ref, counts_ref, lacc_ref, e2_ref):
    i = pl.program_id(0)
    nb = pl.num_programs(0)
    z = z_ref[...]
    e = emb_ref[...]
    k = e.shape[0]
    z2 = jnp.sum(z * z, axis=1, keepdims=True)

    @pl.when(i == 0)
    def _():
        e2_ref[...] = jnp.sum(e * e, axis=1)[None, :]

    e2 = e2_ref[...]
    prodm2 = lax.dot_general(-2.0 * z, e, (((1,), (1,)), ((), ())),
                             preferred_element_type=jnp.float32)
    d = (z2 + e2) + prodm2
    dmin = jnp.min(d, axis=1, keepdims=True)
    iota = lax.broadcasted_iota(jnp.int32, d.shape, 1)
    idx2 = jnp.min(jnp.where(d == dmin, iota, k), axis=1, keepdims=True)
    idx_ref[0, 0, :] = idx2[:, 0]

    onehot = jnp.where(iota == idx2, 1.0, 0.0)
    csum = jnp.sum(onehot, axis=0, keepdims=True)
    cprev = jnp.where(i == 0, jnp.zeros_like(counts_ref[...]), counts_ref[...])
    counts = cprev + csum
    counts_ref[...] = counts

    m = mask_ref[0]
    part = jnp.dot(m, dmin, preferred_element_type=jnp.float32)
    lprev = jnp.where(i == 0, jnp.zeros_like(lacc_ref[...]), lacc_ref[...])
    lacc = lprev + part
    lacc_ref[...] = lacc

    @pl.when(i == nb - 1)
    def _():
        loss_ref[...] = (1.0 + _BETA) * lacc * (1.0 / total)
        e_mean = counts * (1.0 / n_tok)
        ent = e_mean * jnp.log(e_mean + 1e-10)
        perp_ref[...] = jnp.exp(-jnp.sum(ent, keepdims=True))


def _tc_call(z_flat, mask3, embedding):
    n_tok, d_dim = z_flat.shape
    k = embedding.shape[0]
    nb = n_tok // _BT
    total = float(n_tok * d_dim)
    body = functools.partial(_tc_body, float(n_tok), total)
    return pl.pallas_call(
        body,
        grid=(nb,),
        in_specs=[
            pl.BlockSpec((_BT, d_dim), lambda i: (i, 0)),
            pl.BlockSpec((1, 1, _BT), lambda i: (i, 0, 0)),
            pl.BlockSpec((k, d_dim), lambda i: (0, 0)),
        ],
        out_specs=[
            pl.BlockSpec((1, 1, _BT), lambda i: (i, 0, 0)),
            pl.BlockSpec((1, 1), lambda i: (0, 0)),
            pl.BlockSpec((1, 1), lambda i: (0, 0)),
        ],
        out_shape=[
            jax.ShapeDtypeStruct((nb, 1, _BT), jnp.int32),
            jax.ShapeDtypeStruct((1, 1), jnp.float32),
            jax.ShapeDtypeStruct((1, 1), jnp.float32),
        ],
        scratch_shapes=[
            pltpu.VMEM((1, k), jnp.float32),
            pltpu.VMEM((1, 1), jnp.float32),
            pltpu.VMEM((1, k), jnp.float32),
        ],
        compiler_params=pltpu.CompilerParams(
            dimension_semantics=("arbitrary",)),
    )(z_flat, mask3, embedding)


_CH = 128


def _sc_gather(embedding, idx2d, n_tok, d_dim):
    info = plsc.get_sparse_core_info()
    nc, ns = info.num_cores, info.num_subcores
    nw = nc * ns
    n_ch = n_tok // (nw * _CH)
    mesh = plsc.VectorSubcoreMesh(core_axis_name="c", subcore_axis_name="s")

    @functools.partial(
        pl.kernel,
        mesh=mesh,
        out_type=jax.ShapeDtypeStruct((n_tok, d_dim), jnp.float32),
        scratch_types=[
            pltpu.VMEM((n_ch, _CH), jnp.int32),
            pltpu.VMEM((_CH, d_dim), jnp.float32),
            pltpu.SemaphoreType.DMA,
        ],
    )
    def k(table_hbm, idx_hbm, out_hbm, idx_v, rows_v, sem):
        wid = lax.axis_index("s") * nc + lax.axis_index("c")
        pltpu.sync_copy(idx_hbm.at[pl.ds(wid * n_ch, n_ch)], idx_v)
        base = wid * (n_ch * _CH)
        for c in range(n_ch):
            pltpu.async_copy(table_hbm.at[idx_v.at[c]], rows_v, sem).wait()
            pltpu.sync_copy(rows_v, out_hbm.at[pl.ds(base + c * _CH, _CH)])

    return k(embedding, idx2d)


def kernel(z, mask, embedding):
    k, d_dim = embedding.shape
    z_flat = z.reshape(-1, d_dim)
    n_tok = z_flat.shape[0]
    nb = n_tok // _BT
    mask3 = mask.reshape(nb, 1, _BT)
    idx3, loss11, perp11 = _tc_call(z_flat, mask3, embedding)
    idx_flat = idx3.reshape(n_tok)
    idx2d = idx3.reshape(n_tok // _CH, _CH)
    z_q = _sc_gather(embedding, idx2d, n_tok, d_dim)
    return (z_q.reshape(z.shape), idx_flat[:, None], loss11[0, 0],
            perp11[0, 0])


# speedup vs baseline: 1.1184x; 1.1184x over previous
import functools

import jax
import jax.numpy as jnp
from jax import lax
from jax.experimental import pallas as pl
from jax.experimental.pallas import tpu as pltpu
from jax.experimental.pallas import tpu_sc as plsc

_BETA = 0.25
_BT = 1024


def _tc_body(n_tok, total, z_ref, mask_ref, emb_ref, idx_ref, loss_ref,
             perp_ref, counts_ref, lacc_ref, e2_ref):
    i = pl.program_id(0)
    nb = pl.num_programs(0)
    z = z_ref[...]
    e = emb_ref[...]
    k = e.shape[0]
    z2 = jnp.sum(z * z, axis=1, keepdims=True)

    @pl.when(i == 0)
    def _():
        e2_ref[...] = jnp.sum(e * e, axis=1)[None, :]

    e2 = e2_ref[...]
    prodm2 = lax.dot_general(-2.0 * z, e, (((1,), (1,)), ((), ())),
                             preferred_element_type=jnp.float32)
    d = (z2 + e2) + prodm2
    dmin = jnp.min(d, axis=1, keepdims=True)
    iota = lax.broadcasted_iota(jnp.int32, d.shape, 1)
    idx2 = jnp.min(jnp.where(d == dmin, iota, k), axis=1, keepdims=True)
    idx_ref[0, 0, :] = idx2[:, 0]

    onehot = jnp.where(iota == idx2, 1.0, 0.0)
    ones_row = jnp.ones((1, d.shape[0]), jnp.float32)
    csum = jnp.dot(ones_row, onehot,
                   preferred_element_type=jnp.float32)
    cprev = jnp.where(i == 0, jnp.zeros_like(counts_ref[...]), counts_ref[...])
    counts = cprev + csum
    counts_ref[...] = counts

    m = mask_ref[0]
    part = jnp.dot(m, dmin, preferred_element_type=jnp.float32)
    lprev = jnp.where(i == 0, jnp.zeros_like(lacc_ref[...]), lacc_ref[...])
    lacc = lprev + part
    lacc_ref[...] = lacc

    @pl.when(i == nb - 1)
    def _():
        loss_ref[...] = (1.0 + _BETA) * lacc * (1.0 / total)
        e_mean = counts * (1.0 / n_tok)
        ent = e_mean * jnp.log(e_mean + 1e-10)
        perp_ref[...] = jnp.exp(-jnp.sum(ent, keepdims=True))


def _tc_call(z_flat, mask3, embedding):
    n_tok, d_dim = z_flat.shape
    k = embedding.shape[0]
    nb = n_tok // _BT
    total = float(n_tok * d_dim)
    body = functools.partial(_tc_body, float(n_tok), total)
    return pl.pallas_call(
        body,
        grid=(nb,),
        in_specs=[
            pl.BlockSpec((_BT, d_dim), lambda i: (i, 0)),
            pl.BlockSpec((1, 1, _BT), lambda i: (i, 0, 0)),
            pl.BlockSpec((k, d_dim), lambda i: (0, 0)),
        ],
        out_specs=[
            pl.BlockSpec((1, 1, _BT), lambda i: (i, 0, 0)),
            pl.BlockSpec((1, 1), lambda i: (0, 0)),
            pl.BlockSpec((1, 1), lambda i: (0, 0)),
        ],
        out_shape=[
            jax.ShapeDtypeStruct((nb, 1, _BT), jnp.int32),
            jax.ShapeDtypeStruct((1, 1), jnp.float32),
            jax.ShapeDtypeStruct((1, 1), jnp.float32),
        ],
        scratch_shapes=[
            pltpu.VMEM((1, k), jnp.float32),
            pltpu.VMEM((1, 1), jnp.float32),
            pltpu.VMEM((1, k), jnp.float32),
        ],
        compiler_params=pltpu.CompilerParams(
            dimension_semantics=("arbitrary",)),
    )(z_flat, mask3, embedding)


_CH = 128


def _sc_gather(embedding, idx2d, n_tok, d_dim):
    info = plsc.get_sparse_core_info()
    nc, ns = info.num_cores, info.num_subcores
    nw = nc * ns
    n_ch = n_tok // (nw * _CH)
    mesh = plsc.VectorSubcoreMesh(core_axis_name="c", subcore_axis_name="s")

    @functools.partial(
        pl.kernel,
        mesh=mesh,
        out_type=jax.ShapeDtypeStruct((n_tok, d_dim), jnp.float32),
        scratch_types=[
            pltpu.VMEM((n_ch, _CH), jnp.int32),
            pltpu.VMEM((_CH, d_dim), jnp.float32),
            pltpu.SemaphoreType.DMA,
        ],
    )
    def k(table_hbm, idx_hbm, out_hbm, idx_v, rows_v, sem):
        wid = lax.axis_index("s") * nc + lax.axis_index("c")
        pltpu.sync_copy(idx_hbm.at[pl.ds(wid * n_ch, n_ch)], idx_v)
        base = wid * (n_ch * _CH)
        for c in range(n_ch):
            pltpu.async_copy(table_hbm.at[idx_v.at[c]], rows_v, sem).wait()
            pltpu.sync_copy(rows_v, out_hbm.at[pl.ds(base + c * _CH, _CH)])

    return k(embedding, idx2d)


def kernel(z, mask, embedding):
    k, d_dim = embedding.shape
    z_flat = z.reshape(-1, d_dim)
    n_tok = z_flat.shape[0]
    nb = n_tok // _BT
    mask3 = mask.reshape(nb, 1, _BT)
    idx3, loss11, perp11 = _tc_call(z_flat, mask3, embedding)
    idx_flat = idx3.reshape(n_tok)
    idx2d = idx3.reshape(n_tok // _CH, _CH)
    z_q = _sc_gather(embedding, idx2d, n_tok, d_dim)
    return (z_q.reshape(z.shape), idx_flat[:, None], loss11[0, 0],
            perp11[0, 0])
